# Initial kernel scaffold; baseline (speedup 1.0000x reference)
import jax
import jax.numpy as jnp
from jax.experimental import pallas as pl


def kernel(edge_index, x, e, pe, pe_W, pe_b, e1_W, e1_b, e2_W, e2_b, A_W, A_b, B_W, B_b, C_W, C_b, D_W, D_b, Ew_W, Ew_b, bn_h_g, bn_h_b, bn_e_g, bn_e_b, p1_W, p1_b, p2_W, p2_b):
    raise NotImplementedError("write your pallas kernel here")


# speedup vs baseline: 2.3934x; 2.3934x over previous
import functools

import jax
import jax.numpy as jnp
from jax import lax
from jax.experimental import pallas as pl
from jax.experimental.pallas import tpu as pltpu
from jax.experimental.pallas import tpu_sc as plsc

_N = 10000
_E = 160000
_H = 128
_L = 4

_NC = 2
_NS = 16
_W = _NC * _NS

_C = 128
_NCH = _E // _C
_CPW = -(-_NCH // _W)
_CPS = -(-_NCH // _NS)
_RPS = 640
_NP = _RPS * _NS

_BE = 1000
_BN = 1000
_GE = _E // _BE
_GN = _N // _BN


def _f32(shape):
    return jax.ShapeDtypeStruct(shape, jnp.float32)


def _sc_mesh():
    return plsc.VectorSubcoreMesh(
        core_axis_name="c", subcore_axis_name="s", num_cores=_NC,
        num_subcores=_NS)


@functools.cache
def _gather3():
    @functools.partial(
        pl.kernel,
        out_type=(_f32((_E, _H)), _f32((_E, _H)), _f32((_E, _H))),
        mesh=_sc_mesh(),
        scratch_types=[
            pltpu.VMEM((_C,), jnp.int32),
            pltpu.VMEM((_C,), jnp.int32),
            pltpu.VMEM((_C, _H), jnp.float32),
            pltpu.VMEM((_C, _H), jnp.float32),
            pltpu.VMEM((_C, _H), jnp.float32),
            pltpu.SemaphoreType.DMA,
            pltpu.SemaphoreType.DMA,
            pltpu.SemaphoreType.DMA,
        ],
    )
    def k(src_h, dst_h, dh_h, ew_h, bh_h, dhs_o, ehd_o, bhs_o,
          srcv, dstv, b1, b2, b3, s1, s2, s3):
        wid = lax.axis_index("s") * _NC + lax.axis_index("c")

        def body(i, carry):
            ci = wid * _CPW + i

            @pl.when(ci < _NCH)
            def _():
                base = ci * _C
                pltpu.sync_copy(src_h.at[pl.ds(base, _C)], srcv)
                pltpu.sync_copy(dst_h.at[pl.ds(base, _C)], dstv)
                c1 = pltpu.async_copy(dh_h.at[srcv], b1, s1)
                c2 = pltpu.async_copy(ew_h.at[dstv], b2, s2)
                c3 = pltpu.async_copy(bh_h.at[srcv], b3, s3)
                c1.wait()
                pltpu.sync_copy(b1, dhs_o.at[pl.ds(base, _C)])
                c2.wait()
                pltpu.sync_copy(b2, ehd_o.at[pl.ds(base, _C)])
                c3.wait()
                pltpu.sync_copy(b3, bhs_o.at[pl.ds(base, _C)])

            return carry

        lax.fori_loop(0, _CPW, body, 0)

    return k


@functools.cache
def _gather2():
    @functools.partial(
        pl.kernel,
        out_type=(_f32((_E, _H)), _f32((_E, _H))),
        mesh=_sc_mesh(),
        scratch_types=[
            pltpu.VMEM((_C,), jnp.int32),
            pltpu.VMEM((_C,), jnp.int32),
            pltpu.VMEM((_C, _H), jnp.float32),
            pltpu.VMEM((_C, _H), jnp.float32),
            pltpu.SemaphoreType.DMA,
            pltpu.SemaphoreType.DMA,
        ],
    )
    def k(src_h, dst_h, h_h, hs_o, hd_o, srcv, dstv, b1, b2, s1, s2):
        wid = lax.axis_index("s") * _NC + lax.axis_index("c")

        def body(i, carry):
            ci = wid * _CPW + i

            @pl.when(ci < _NCH)
            def _():
                base = ci * _C
                pltpu.sync_copy(src_h.at[pl.ds(base, _C)], srcv)
                pltpu.sync_copy(dst_h.at[pl.ds(base, _C)], dstv)
                c1 = pltpu.async_copy(h_h.at[srcv], b1, s1)
                c2 = pltpu.async_copy(h_h.at[dstv], b2, s2)
                c1.wait()
                pltpu.sync_copy(b1, hs_o.at[pl.ds(base, _C)])
                c2.wait()
                pltpu.sync_copy(b2, hd_o.at[pl.ds(base, _C)])

            return carry

        lax.fori_loop(0, _CPW, body, 0)

    return k


@functools.cache
def _scatter2():
    @functools.partial(
        pl.kernel,
        out_type=(_f32((_N, _H)), _f32((_N, _H))),
        mesh=_sc_mesh(),
        scratch_types=[
            pltpu.VMEM((_C,), jnp.int32),
            pltpu.VMEM((_C, _H), jnp.float32),
            pltpu.VMEM_SHARED((_NP, _H), jnp.float32),
        ],
    )
    def k(msg_h, sig_h, dst_h, zeros_h, num_o, den_o, idxv, valv, accum):
        c = lax.axis_index("c")
        s = lax.axis_index("s")

        pltpu.sync_copy(zeros_h, accum.at[pl.ds(s * _RPS, _RPS)])
        plsc.subcore_barrier()

        def body(i, carry):
            ci = s * _CPS + i

            @pl.when(ci < _NCH)
            def _():
                base = ci * _C
                pltpu.sync_copy(dst_h.at[pl.ds(base, _C)], idxv)

                @pl.when(c == 0)
                def _():
                    pltpu.sync_copy(msg_h.at[pl.ds(base, _C)], valv)

                @pl.when(c == 1)
                def _():
                    pltpu.sync_copy(sig_h.at[pl.ds(base, _C)], valv)

                pltpu.sync_copy(valv, accum.at[idxv], add=True)

            return carry

        lax.fori_loop(0, _CPS, body, 0)
        plsc.subcore_barrier()

        last = _N - _RPS * (_NS - 1)

        @pl.when(jnp.logical_and(c == 0, s < _NS - 1))
        def _():
            pltpu.sync_copy(accum.at[pl.ds(s * _RPS, _RPS)],
                            num_o.at[pl.ds(s * _RPS, _RPS)])

        @pl.when(jnp.logical_and(c == 0, s == _NS - 1))
        def _():
            pltpu.sync_copy(accum.at[pl.ds((_NS - 1) * _RPS, last)],
                            num_o.at[pl.ds((_NS - 1) * _RPS, last)])

        @pl.when(jnp.logical_and(c == 1, s < _NS - 1))
        def _():
            pltpu.sync_copy(accum.at[pl.ds(s * _RPS, _RPS)],
                            den_o.at[pl.ds(s * _RPS, _RPS)])

        @pl.when(jnp.logical_and(c == 1, s == _NS - 1))
        def _():
            pltpu.sync_copy(accum.at[pl.ds((_NS - 1) * _RPS, last)],
                            den_o.at[pl.ds((_NS - 1) * _RPS, last)])

    return k


def _row_spec(b, w):
    return pl.BlockSpec((b, w), lambda i: (i, 0))


def _full_spec(r, w):
    return pl.BlockSpec((r, w), lambda i: (0, 0))


def _h0_body(pe_r, w_r, b_r, out_r):
    out_r[...] = (jnp.dot(pe_r[...], w_r[...],
                          preferred_element_type=jnp.float32) + b_r[...])


@functools.cache
def _h0_call():
    return pl.pallas_call(
        _h0_body,
        grid=(_GN,),
        in_specs=[_row_spec(_BN, 10), _full_spec(10, _H), _full_spec(1, _H)],
        out_specs=_row_spec(_BN, _H),
        out_shape=_f32((_N, _H)),
    )


def _eh0_body(e_r, w1_r, b1_r, w2_r, b2_r, out_r):
    t = jnp.dot(e_r[...], w1_r[...], preferred_element_type=jnp.float32)
    t = jnp.maximum(t + b1_r[...], 0.0)
    out_r[...] = (jnp.dot(t, w2_r[...], preferred_element_type=jnp.float32)
                  + b2_r[...])


@functools.cache
def _eh0_call():
    return pl.pallas_call(
        _eh0_body,
        grid=(_GE,),
        in_specs=[_row_spec(_BE, 16), _full_spec(16, 16), _full_spec(1, 16),
                  _full_spec(16, _H), _full_spec(1, _H)],
        out_specs=_row_spec(_BE, _H),
        out_shape=_f32((_E, _H)),
    )


def _node_mm_body(h_r, w_r, b_r, a_o, b_o, d_o, e_o):
    r = jnp.dot(h_r[...], w_r[...], preferred_element_type=jnp.float32)
    r = r + b_r[...]
    a_o[...] = r[:, 0 * _H:1 * _H]
    b_o[...] = r[:, 1 * _H:2 * _H]
    d_o[...] = r[:, 2 * _H:3 * _H]
    e_o[...] = r[:, 3 * _H:4 * _H]


@functools.cache
def _node_mm_call():
    return pl.pallas_call(
        _node_mm_body,
        grid=(_GN,),
        in_specs=[_row_spec(_BN, _H), _full_spec(_H, 4 * _H),
                  _full_spec(1, 4 * _H)],
        out_specs=tuple(_row_spec(_BN, _H) for _ in range(4)),
        out_shape=tuple(_f32((_N, _H)) for _ in range(4)),
    )


def _edge1_body(eh_r, cw_r, cb_r, dhs_r, ehd_r, enew_o, stats_o):
    i = pl.program_id(0)
    v = jnp.dot(eh_r[...], cw_r[...], preferred_element_type=jnp.float32)
    v = v + cb_r[...] + dhs_r[...] + ehd_r[...]
    enew_o[...] = v
    st = jnp.concatenate(
        [jnp.sum(v, axis=0, keepdims=True),
         jnp.sum(v * v, axis=0, keepdims=True),
         jnp.zeros((6, _H), jnp.float32)], axis=0)

    @pl.when(i == 0)
    def _():
        stats_o[...] = st

    @pl.when(i > 0)
    def _():
        stats_o[...] += st


@functools.cache
def _edge1_call():
    return pl.pallas_call(
        _edge1_body,
        grid=(_GE,),
        in_specs=[_row_spec(_BE, _H), _full_spec(_H, _H), _full_spec(1, _H),
                  _row_spec(_BE, _H), _row_spec(_BE, _H)],
        out_specs=(_row_spec(_BE, _H), _full_spec(8, _H)),
        out_shape=(_f32((_E, _H)), _f32((8, _H))),
    )


def _edge2_body(eh_r, enew_r, bhs_r, st_r, g_r, b_r, ehn_o, sig_o, msg_o):
    st = st_r[...]
    m = st[0:1, :] * (1.0 / _E)
    var = st[1:2, :] * (1.0 / _E) - m * m
    inv = lax.rsqrt(var + 1e-5)
    xb = g_r[...] * (enew_r[...] - m) * inv + b_r[...]
    ehn = eh_r[...] + jnp.maximum(xb, 0.0)
    sig = jax.nn.sigmoid(ehn)
    ehn_o[...] = ehn
    sig_o[...] = sig
    msg_o[...] = sig * bhs_r[...]


@functools.cache
def _edge2_call():
    return pl.pallas_call(
        _edge2_body,
        grid=(_GE,),
        in_specs=[_row_spec(_BE, _H), _row_spec(_BE, _H), _row_spec(_BE, _H),
                  _full_spec(8, _H), _full_spec(1, _H), _full_spec(1, _H)],
        out_specs=tuple(_row_spec(_BE, _H) for _ in range(3)),
        out_shape=tuple(_f32((_E, _H)) for _ in range(3)),
    )


def _node1_body(ah_r, num_r, den_r, t_o, stats_o):
    i = pl.program_id(0)
    v = ah_r[...] + num_r[...] / (den_r[...] + 1e-6)
    t_o[...] = v
    st = jnp.concatenate(
        [jnp.sum(v, axis=0, keepdims=True),
         jnp.sum(v * v, axis=0, keepdims=True),
         jnp.zeros((6, _H), jnp.float32)], axis=0)

    @pl.when(i == 0)
    def _():
        stats_o[...] = st

    @pl.when(i > 0)
    def _():
        stats_o[...] += st


@functools.cache
def _node1_call():
    return pl.pallas_call(
        _node1_body,
        grid=(_GN,),
        in_specs=[_row_spec(_BN, _H)] * 3,
        out_specs=(_row_spec(_BN, _H), _full_spec(8, _H)),
        out_shape=(_f32((_N, _H)), _f32((8, _H))),
    )


def _node2_body(h_r, t_r, st_r, g_r, b_r, h_o):
    st = st_r[...]
    m = st[0:1, :] * (1.0 / _N)
    var = st[1:2, :] * (1.0 / _N) - m * m
    inv = lax.rsqrt(var + 1e-5)
    xb = g_r[...] * (t_r[...] - m) * inv + b_r[...]
    h_o[...] = h_r[...] + jnp.maximum(xb, 0.0)


@functools.cache
def _node2_call():
    return pl.pallas_call(
        _node2_body,
        grid=(_GN,),
        in_specs=[_row_spec(_BN, _H), _row_spec(_BN, _H), _full_spec(8, _H),
                  _full_spec(1, _H), _full_spec(1, _H)],
        out_specs=_row_spec(_BN, _H),
        out_shape=_f32((_N, _H)),
    )


def _final_body(hs_r, hd_r, eh_r, pa_r, pb_r, pc_r, p1b_r, p2_r, p2b_r,
                out_o):
    z = jnp.dot(hs_r[...], pa_r[...], preferred_element_type=jnp.float32)
    z = z + jnp.dot(hd_r[...], pb_r[...], preferred_element_type=jnp.float32)
    z = z + jnp.dot(eh_r[...], pc_r[...], preferred_element_type=jnp.float32)
    z = jnp.maximum(z + p1b_r[...], 0.0)
    out_o[...] = (jnp.dot(z, p2_r[...], preferred_element_type=jnp.float32)
                  + p2b_r[...])


@functools.cache
def _final_call():
    return pl.pallas_call(
        _final_body,
        grid=(_GE,),
        in_specs=[_row_spec(_BE, _H)] * 3
        + [_full_spec(_H, _H)] * 3
        + [_full_spec(1, _H), _full_spec(_H, 1), _full_spec(1, 1)],
        out_specs=_row_spec(_BE, 1),
        out_shape=_f32((_E, 1)),
    )


def kernel(edge_index, x, e, pe, pe_W, pe_b, e1_W, e1_b, e2_W, e2_b,
           A_W, A_b, B_W, B_b, C_W, C_b, D_W, D_b, Ew_W, Ew_b,
           bn_h_g, bn_h_b, bn_e_g, bn_e_b, p1_W, p1_b, p2_W, p2_b):
    src = edge_index[0]
    dst = edge_index[1]

    h = _h0_call()(pe, pe_W, pe_b.reshape(1, _H))
    eh = _eh0_call()(e, e1_W, e1_b.reshape(1, 16), e2_W, e2_b.reshape(1, _H))

    W4 = jnp.concatenate([A_W, B_W, D_W, Ew_W], axis=2)
    b4 = jnp.concatenate([A_b, B_b, D_b, Ew_b], axis=1)
    zeros_n = jnp.zeros((_RPS, _H), jnp.float32)

    for l in range(_L):
        Ah, Bh, Dh, Eh = _node_mm_call()(h, W4[l], b4[l].reshape(1, 4 * _H))
        dhs, ehd, bhs = _gather3()(src, dst, Dh, Eh, Bh)
        enew, est = _edge1_call()(eh, C_W[l], C_b[l].reshape(1, _H), dhs, ehd)
        eh, sig, msg = _edge2_call()(eh, enew, bhs, est,
                                     bn_e_g[l].reshape(1, _H),
                                     bn_e_b[l].reshape(1, _H))
        num, den = _scatter2()(msg, sig, dst, zeros_n)
        t, nst = _node1_call()(Ah, num, den)
        h = _node2_call()(h, t, nst, bn_h_g[l].reshape(1, _H),
                          bn_h_b[l].reshape(1, _H))

    hs, hd = _gather2()(src, dst, h)
    scores = _final_call()(hs, hd, eh,
                           p1_W[0 * _H:1 * _H], p1_W[1 * _H:2 * _H],
                           p1_W[2 * _H:3 * _H], p1_b.reshape(1, _H),
                           p2_W, p2_b.reshape(1, 1))
    return scores
